# Initial kernel scaffold; baseline (speedup 1.0000x reference)
import jax
import jax.numpy as jnp
from jax.experimental import pallas as pl


def kernel(tgt_seq, pos_w_embedding, pos_h_embedding):
    raise NotImplementedError("write your pallas kernel here")


# speedup vs baseline: 4.9244x; 4.9244x over previous
import functools

import jax
import jax.numpy as jnp
from jax import lax
from jax.experimental import pallas as pl
from jax.experimental.pallas import tpu as pltpu
from jax.experimental.pallas import tpu_sc as plsc

NC = 2
NS = 16
NW = NC * NS

HALF = 256
DM = 2 * HALF
TROWS = 768
RPW = TROWS // NS
CH = 64


def _body(idx_hbm, pw_hbm, ph_hbm, out_hbm, ct_hbm,
          xidx, yidx, bufw, bufh, idxb, ob0, ob1,
          sw, sh, g0, g1, s0, s1):
    c = lax.axis_index("c")
    s = lax.axis_index("s")
    wid = s * NC + c

    r0 = s * RPW
    for j in range(3):
        t = r0 + j * 16 + lax.iota(jnp.int32, 16)
        a = t - 2
        valid = (t >= 2) & (t < 644)
        xidx[pl.ds(j * 16, 16)] = jnp.where(valid, a & 31, 32)
        yidx[pl.ds(j * 16, 16)] = jnp.where(valid, a >> 5, 20)
    cw = pltpu.async_copy(pw_hbm.at[xidx], bufw, sw)
    ch = pltpu.async_copy(ph_hbm.at[yidx], bufh, sh)
    cw.wait()
    ch.wait()
    ctab = ct_hbm.at[c]
    pltpu.sync_copy(bufw, ctab.at[pl.ds(r0, RPW), pl.ds(0, HALF)])
    pltpu.sync_copy(bufh, ctab.at[pl.ds(r0, RPW), pl.ds(HALF, HALF)])
    plsc.subcore_barrier()

    nchunks = idxb.shape[0]
    row0 = wid * (nchunks * CH)
    pltpu.sync_copy(idx_hbm.at[wid], idxb)

    def gather(k, ob, sem):
        return pltpu.async_copy(ctab.at[idxb.at[k]], ob, sem)

    def store(k, ob, sem):
        return pltpu.async_copy(ob, out_hbm.at[pl.ds(row0 + k * CH, CH)], sem)

    gather(0, ob0, g0)
    gather(1, ob1, g1)

    @pl.loop(0, nchunks - 2, step=2)
    def _(k):
        pltpu.make_async_copy(ctab.at[idxb.at[k]], ob0, g0).wait()
        store(k, ob0, s0)
        pltpu.make_async_copy(ctab.at[idxb.at[k]], ob1, g1).wait()
        store(k + 1, ob1, s1)
        pltpu.make_async_copy(ob0, out_hbm.at[pl.ds(row0, CH)], s0).wait()
        gather(k + 2, ob0, g0)
        pltpu.make_async_copy(ob1, out_hbm.at[pl.ds(row0, CH)], s1).wait()
        gather(k + 3, ob1, g1)

    k = nchunks - 2
    pltpu.make_async_copy(ctab.at[idxb.at[k]], ob0, g0).wait()
    store(k, ob0, s0)
    pltpu.make_async_copy(ctab.at[idxb.at[k]], ob1, g1).wait()
    store(k + 1, ob1, s1)
    pltpu.make_async_copy(ob0, out_hbm.at[pl.ds(row0, CH)], s0).wait()
    pltpu.make_async_copy(ob1, out_hbm.at[pl.ds(row0, CH)], s1).wait()


@jax.jit
def _sc_lookup(idx3d, pw_ext, ph_ext):
    n = idx3d.shape[0] * idx3d.shape[1] * idx3d.shape[2]
    nchunks = n // (NW * CH)
    run = pl.kernel(
        _body,
        out_type=(jax.ShapeDtypeStruct((n, DM), jnp.float32),
                  jax.ShapeDtypeStruct((NC, TROWS, DM), jnp.float32)),
        mesh=plsc.VectorSubcoreMesh(core_axis_name="c", subcore_axis_name="s"),
        scratch_types=[
            pltpu.VMEM((48,), jnp.int32),
            pltpu.VMEM((48,), jnp.int32),
            pltpu.VMEM((48, HALF), jnp.float32),
            pltpu.VMEM((48, HALF), jnp.float32),
            pltpu.VMEM((nchunks, CH), jnp.int32),
            pltpu.VMEM((CH, DM), jnp.float32),
            pltpu.VMEM((CH, DM), jnp.float32),
        ] + [pltpu.SemaphoreType.DMA] * 6,
    )
    out, _ = run(idx3d, pw_ext, ph_ext)
    return out


def kernel(tgt_seq, pos_w_embedding, pos_h_embedding):
    b, seq = tgt_seq.shape
    n = b * seq
    pw_ext = jnp.pad(pos_w_embedding, ((0, 2), (0, 0)))
    ph_ext = jnp.pad(pos_h_embedding, ((0, 2), (0, 0)))
    idx3d = tgt_seq.reshape(NW, n // (NW * CH), CH)
    out = _sc_lookup(idx3d, pw_ext, ph_ext)
    return out.reshape(b, seq, DM)
